# baseline (speedup 1.0000x reference)
import functools

import jax
import jax.numpy as jnp
from jax import lax
from jax.experimental import pallas as pl
from jax.experimental.pallas import tpu as pltpu
from jax.experimental.pallas import tpu_sc as plsc

_SEQ_BLK = 2048
_SC_BATCH = 1
_NC = 2
_NS = 16
_CHUNK = 32768


def _tc_add(x_ref, pos_ref, o_ref):
    o_ref[...] = x_ref[...] + pos_ref[...]


def _tc_call(x, pos, n_batch):
    _, seq_len, d_model = x.shape
    n_s = seq_len // _SEQ_BLK
    return pl.pallas_call(
        _tc_add,
        grid=(n_s, n_batch),
        in_specs=[
            pl.BlockSpec((1, _SEQ_BLK, d_model), lambda s, b: (b, s, 0)),
            pl.BlockSpec((_SEQ_BLK, d_model), lambda s, b: (s, 0)),
        ],
        out_specs=pl.BlockSpec((1, _SEQ_BLK, d_model), lambda s, b: (b, s, 0)),
        out_shape=jax.ShapeDtypeStruct((n_batch, seq_len, d_model), x.dtype),
    )(x, pos)


def _sc_call(x_flat, pos_flat, sc_off, n_sc):
    per_w = n_sc // (_NC * _NS)
    n_chunks = per_w // _CHUNK
    mesh = plsc.VectorSubcoreMesh(core_axis_name="c", subcore_axis_name="s")

    @functools.partial(
        pl.kernel,
        mesh=mesh,
        out_type=jax.ShapeDtypeStruct((n_sc,), jnp.float32),
        scratch_types=[
            pltpu.VMEM((_CHUNK,), jnp.float32),
            pltpu.VMEM((_CHUNK,), jnp.float32),
        ],
    )
    def k(x_hbm, p_hbm, o_hbm, xv, pv):
        wid = lax.axis_index("s") * _NC + lax.axis_index("c")
        base = wid * per_w

        def chunk_body(c, carry):
            off = base + c * _CHUNK
            pltpu.sync_copy(x_hbm.at[pl.ds(sc_off + off, _CHUNK)], xv)
            pltpu.sync_copy(p_hbm.at[pl.ds(off, _CHUNK)], pv)

            def add_body(i, carry2):
                b0 = i * 128
                for j in range(8):
                    sl = pl.ds(b0 + j * 16, 16)
                    xv[sl] = xv[sl] + pv[sl]
                return carry2

            lax.fori_loop(0, _CHUNK // 128, add_body, 0)
            pltpu.sync_copy(xv, o_hbm.at[pl.ds(off, _CHUNK)])
            return carry

        lax.fori_loop(0, n_chunks, chunk_body, 0)

    return k(x_flat, pos_flat)


def kernel(x, pos_table):
    batch, seq_len, d_model = x.shape
    pos = pos_table[:seq_len]
    n_tc = batch - _SC_BATCH

    tc_out = _tc_call(x, pos, n_tc)

    n_sc = _SC_BATCH * seq_len * d_model
    sc_out = _sc_call(
        x.reshape(-1), pos.reshape(-1), n_tc * seq_len * d_model, n_sc
    ).reshape(_SC_BATCH, seq_len, d_model)

    return jnp.concatenate([tc_out, sc_out], axis=0)


# speedup vs baseline: 3.9098x; 3.9098x over previous
import jax
import jax.numpy as jnp
from jax.experimental import pallas as pl
from jax.experimental.pallas import tpu as pltpu

_SEQ_BLK = 2048


def _add_kernel(x_ref, pos_ref, o_ref):
    o_ref[...] = x_ref[...] + pos_ref[...]


def kernel(x, pos_table):
    batch, seq_len, d_model = x.shape
    pos = pos_table[:seq_len]
    n_s = seq_len // _SEQ_BLK
    return pl.pallas_call(
        _add_kernel,
        grid=(n_s, batch),
        in_specs=[
            pl.BlockSpec((1, _SEQ_BLK, d_model), lambda s, b: (b, s, 0)),
            pl.BlockSpec((_SEQ_BLK, d_model), lambda s, b: (s, 0)),
        ],
        out_specs=pl.BlockSpec((1, _SEQ_BLK, d_model), lambda s, b: (b, s, 0)),
        out_shape=jax.ShapeDtypeStruct((batch, seq_len, d_model), x.dtype),
    )(x, pos)


# speedup vs baseline: 3.9151x; 1.0014x over previous
import jax
import jax.numpy as jnp
from jax.experimental import pallas as pl

_SEQ_BLK = 2048


def _add_kernel(x_ref, pos_ref, o_ref):
    o_ref[...] = x_ref[...] + pos_ref[...]


def kernel(x, pos_table):
    batch, seq_len, d_model = x.shape
    pos = pos_table[:seq_len]
    n_s = seq_len // _SEQ_BLK
    return pl.pallas_call(
        _add_kernel,
        grid=(n_s, batch),
        in_specs=[
            pl.BlockSpec((1, _SEQ_BLK, d_model), lambda s, b: (b, s, 0)),
            pl.BlockSpec((_SEQ_BLK, d_model), lambda s, b: (s, 0)),
        ],
        out_specs=pl.BlockSpec((1, _SEQ_BLK, d_model), lambda s, b: (b, s, 0)),
        out_shape=jax.ShapeDtypeStruct((batch, seq_len, d_model), x.dtype),
    )(x, pos)
